# Initial kernel scaffold; baseline (speedup 1.0000x reference)
import jax
import jax.numpy as jnp
from jax.experimental import pallas as pl


def kernel(node_features, edge_attr, edge_weights, edge_index, We1, be1, We2, be2, Wge1, bge1, Wge2, bge2, Wel, Wn1, bn1, Wn2, bn2, Wgn1, bgn1, Wgn2, bgn2, Wnl):
    raise NotImplementedError("write your pallas kernel here")


# speedup vs baseline: 2.6678x; 2.6678x over previous
import functools

import jax
import jax.numpy as jnp
from jax import lax
from jax.experimental import pallas as pl
from jax.experimental.pallas import tpu as pltpu
from jax.experimental.pallas import tpu_sc as plsc

NC, NS = 2, 16
NW = NC * NS
CG = 400
CS = 200
NB = 2000
EB = 2000


def _silu(x):
    return x * jax.nn.sigmoid(x)


def _proj_body(nf_ref, w_ref, ps_ref, pd_ref):
    nf = nf_ref[...]
    ps_ref[...] = jnp.dot(nf, w_ref[0], preferred_element_type=jnp.float32)
    pd_ref[...] = jnp.dot(nf, w_ref[1], preferred_element_type=jnp.float32)


def _edge_body(gs_ref, gd_ref, ea_ref, ew_ref, w16_ref, b16_ref, w128_ref,
               b128_ref, wel_ref, wnl_ref, ea2_ref, feat_ref):
    gs = gs_ref[...]
    gd = gd_ref[...]
    ea = ea_ref[...]
    ew = ew_ref[...]

    def mm(x, w):
        return jnp.dot(x, w, preferred_element_type=jnp.float32)

    h1 = _silu(gs[:, 0:16] + gd[:, 0:16] + mm(ea, w16_ref[0]) + b16_ref[0])
    g1 = _silu(gs[:, 16:32] + gd[:, 16:32] + mm(ea, w16_ref[1]) + b16_ref[1])
    h2 = _silu(mm(h1, w16_ref[2]) + b16_ref[2])
    g2 = jax.nn.sigmoid(mm(g1, w16_ref[3]) + b16_ref[3])
    ea2 = ea + h2 * g2 * mm(ew, wel_ref[...])
    ea2_ref[...] = ea2
    hn1 = _silu(gs[:, 32:48] + gd[:, 32:48] + mm(ea2, w16_ref[4]) + b16_ref[4])
    gn1 = _silu(gs[:, 48:64] + gd[:, 48:64] + mm(ea2, w16_ref[5]) + b16_ref[5])
    hn2 = _silu(mm(hn1, w128_ref[0]) + b128_ref[0])
    gn2 = jax.nn.sigmoid(mm(gn1, w128_ref[1]) + b128_ref[1])
    feat_ref[...] = hn2 * gn2 * mm(ew, wnl_ref[...])


def _comb_body(p_ref, o_ref):
    o_ref[...] = p_ref[0] + p_ref[1]


def kernel(node_features, edge_attr, edge_weights, edge_index, We1, be1, We2,
           be2, Wge1, bge1, Wge2, bge2, Wel, Wn1, bn1, Wn2, bn2, Wgn1, bgn1,
           Wgn2, bgn2, Wnl):
    N, DN = node_features.shape
    E, DE = edge_attr.shape
    f32 = jnp.float32
    src = edge_index[0]
    dst = edge_index[1]

    wsrc = jnp.concatenate([We1[:DN], Wge1[:DN], Wn1[:DN], Wgn1[:DN]], axis=1)
    wdst = jnp.concatenate([We1[DN:2 * DN], Wge1[DN:2 * DN], Wn1[DN:2 * DN],
                            Wgn1[DN:2 * DN]], axis=1)
    wstack = jnp.stack([wsrc, wdst])
    w16 = jnp.stack([We1[2 * DN:], Wge1[2 * DN:], We2, Wge2,
                     Wn1[2 * DN:], Wgn1[2 * DN:]])
    b16 = jnp.stack([be1, bge1, be2, bge2, bn1, bgn1])
    w128 = jnp.stack([Wn2, Wgn2])
    b128 = jnp.stack([bn2, bgn2])

    psrc, pdst = pl.pallas_call(
        _proj_body,
        grid=(N // NB,),
        in_specs=[pl.BlockSpec((NB, DN), lambda i: (i, 0)),
                  pl.BlockSpec((2, DN, 64), lambda i: (0, 0, 0))],
        out_specs=[pl.BlockSpec((NB, 64), lambda i: (i, 0)),
                   pl.BlockSpec((NB, 64), lambda i: (i, 0))],
        out_shape=[jax.ShapeDtypeStruct((N, 64), f32),
                   jax.ShapeDtypeStruct((N, 64), f32)],
    )(node_features, wstack)

    mesh = plsc.VectorSubcoreMesh(core_axis_name="c", subcore_axis_name="s")

    rows_g = E // 16
    nch_g = rows_g // CG

    @functools.partial(
        pl.kernel,
        out_type=[jax.ShapeDtypeStruct((E, 64), f32),
                  jax.ShapeDtypeStruct((E, 64), f32)],
        mesh=mesh,
        scratch_types=[pltpu.VMEM((CG,), jnp.int32),
                       pltpu.VMEM((CG, 64), f32),
                       pltpu.SemaphoreType.DMA],
        compiler_params=pltpu.CompilerParams(use_tc_tiling_on_sc=False),
    )
    def _gather_k(ps_hbm, pd_hbm, src_hbm, dst_hbm, gs_hbm, gd_hbm,
                  idx_v, rows_v, sem):
        cid = lax.axis_index("c")
        sid = lax.axis_index("s")
        wid = sid * NC + cid
        w16_ = wid % 16
        base = w16_ * rows_g

        def run(table, idxarr, out):
            def chunk(k, carry):
                off = base + k * CG
                pltpu.sync_copy(idxarr.at[pl.ds(off, CG)], idx_v)
                pltpu.async_copy(table.at[idx_v], rows_v, sem).wait()
                pltpu.sync_copy(rows_v, out.at[pl.ds(off, CG)])
                return carry
            lax.fori_loop(0, nch_g, chunk, 0)

        @pl.when(wid < 16)
        def _():
            run(ps_hbm, src_hbm, gs_hbm)

        @pl.when(wid >= 16)
        def _():
            run(pd_hbm, dst_hbm, gd_hbm)

    gs, gd = _gather_k(psrc, pdst, src, dst)

    DEG = edge_weights.shape[1]
    full = lambda *s: pl.BlockSpec(s, lambda i: tuple(0 for _ in s))
    ea2, feat = pl.pallas_call(
        _edge_body,
        grid=(E // EB,),
        in_specs=[pl.BlockSpec((EB, 64), lambda i: (i, 0)),
                  pl.BlockSpec((EB, 64), lambda i: (i, 0)),
                  pl.BlockSpec((EB, DE), lambda i: (i, 0)),
                  pl.BlockSpec((EB, DEG), lambda i: (i, 0)),
                  full(6, DE, DE), full(6, DE), full(2, DE, DN),
                  full(2, DN), full(DEG, DE), full(DEG, DN)],
        out_specs=[pl.BlockSpec((EB, DE), lambda i: (i, 0)),
                   pl.BlockSpec((EB, DN), lambda i: (i, 0))],
        out_shape=[jax.ShapeDtypeStruct((E, DE), f32),
                   jax.ShapeDtypeStruct((E, DN), f32)],
        compiler_params=pltpu.CompilerParams(
            dimension_semantics=("arbitrary",)),
    )(gs, gd, edge_attr, edge_weights, w16, b16, w128, b128, Wel, Wnl)

    rows_s = E // NW
    nch_s = rows_s // CS
    NT = N // NS
    init = jnp.stack([node_features, jnp.zeros((N, DN), f32)])

    @functools.partial(
        pl.kernel,
        out_type=jax.ShapeDtypeStruct((2, N, DN), f32),
        mesh=mesh,
        scratch_types=[pltpu.VMEM((CS,), jnp.int32),
                       pltpu.VMEM((CS, DN), f32),
                       pltpu.VMEM_SHARED((N, DN), f32)],
        compiler_params=pltpu.CompilerParams(use_tc_tiling_on_sc=False),
    )
    def _scatter_k(feat_hbm, src_hbm, init_hbm, out_hbm, idx_v, rows_v, acc):
        cid = lax.axis_index("c")
        sid = lax.axis_index("s")
        wid = sid * NC + cid
        pltpu.sync_copy(init_hbm.at[cid, pl.ds(sid * NT, NT)],
                        acc.at[pl.ds(sid * NT, NT)])
        plsc.subcore_barrier()
        base = wid * rows_s

        def chunk(k, carry):
            off = base + k * CS
            pltpu.sync_copy(src_hbm.at[pl.ds(off, CS)], idx_v)
            pltpu.sync_copy(feat_hbm.at[pl.ds(off, CS)], rows_v)
            pltpu.sync_copy(rows_v, acc.at[idx_v], add=True)
            return carry
        lax.fori_loop(0, nch_s, chunk, 0)
        plsc.subcore_barrier()
        pltpu.sync_copy(acc.at[pl.ds(sid * NT, NT)],
                        out_hbm.at[cid, pl.ds(sid * NT, NT)])

    parts = _scatter_k(feat, src, init)

    node2 = pl.pallas_call(
        _comb_body,
        grid=(N // NB,),
        in_specs=[pl.BlockSpec((2, NB, DN), lambda i: (0, i, 0))],
        out_specs=pl.BlockSpec((NB, DN), lambda i: (i, 0)),
        out_shape=jax.ShapeDtypeStruct((N, DN), f32),
    )(parts)

    return (node2, ea2)


# speedup vs baseline: 2.7004x; 1.0122x over previous
import functools

import jax
import jax.numpy as jnp
from jax import lax
from jax.experimental import pallas as pl
from jax.experimental.pallas import tpu as pltpu
from jax.experimental.pallas import tpu_sc as plsc

NC, NS = 2, 16
NW = NC * NS
CG = 400
CS = 200
NB = 2000
EB = 2000
SEG = 5

_LOG2E = 1.4426950408889634


def _sig(x):
    return 1.0 / (1.0 + jnp.exp2(x * (-_LOG2E)))


def _silu(x):
    return x * _sig(x)


def _proj_body(nf_ref, w_ref, ps_ref, pd_ref):
    nf = nf_ref[...]
    ps_ref[...] = jnp.dot(nf, w_ref[0], preferred_element_type=jnp.float32)
    pd_ref[...] = jnp.dot(nf, w_ref[1], preferred_element_type=jnp.float32)


def _edge_body(gs_ref, gd_ref, ea_ref, ew_ref, w16_ref, b16_ref, w128_ref,
               b128_ref, wel_ref, wnl_ref, ea2_ref, feat_ref):
    gs = gs_ref[...]
    gd = gd_ref[...]
    ea = ea_ref[...]
    ew = ew_ref[...]

    def mm(x, w):
        return jnp.dot(x, w, preferred_element_type=jnp.float32)

    h1 = _silu(gs[:, 0:16] + gd[:, 0:16] + mm(ea, w16_ref[0]) + b16_ref[0])
    g1 = _silu(gs[:, 16:32] + gd[:, 16:32] + mm(ea, w16_ref[1]) + b16_ref[1])
    h2 = _silu(mm(h1, w16_ref[2]) + b16_ref[2])
    g2 = _sig(mm(g1, w16_ref[3]) + b16_ref[3])
    ea2 = ea + h2 * g2 * mm(ew, wel_ref[...])
    ea2_ref[...] = ea2
    hn1 = _silu(gs[:, 32:48] + gd[:, 32:48] + mm(ea2, w16_ref[4]) + b16_ref[4])
    gn1 = _silu(gs[:, 48:64] + gd[:, 48:64] + mm(ea2, w16_ref[5]) + b16_ref[5])
    hn2 = _silu(mm(hn1, w128_ref[0]) + b128_ref[0])
    gn2 = _sig(mm(gn1, w128_ref[1]) + b128_ref[1])
    feat_ref[...] = hn2 * gn2 * mm(ew, wnl_ref[...])


def _comb_body(nf_ref, *refs):
    prefs = refs[:-1]
    o_ref = refs[-1]
    acc = nf_ref[...]
    for p in prefs:
        acc = acc + p[0] + p[1]
    o_ref[...] = acc


def kernel(node_features, edge_attr, edge_weights, edge_index, We1, be1, We2,
           be2, Wge1, bge1, Wge2, bge2, Wel, Wn1, bn1, Wn2, bn2, Wgn1, bgn1,
           Wgn2, bgn2, Wnl):
    N, DN = node_features.shape
    E, DE = edge_attr.shape
    DEG = edge_weights.shape[1]
    f32 = jnp.float32
    src = edge_index[0]
    dst = edge_index[1]
    ES = E // SEG

    wsrc = jnp.concatenate([We1[:DN], Wge1[:DN], Wn1[:DN], Wgn1[:DN]], axis=1)
    wdst = jnp.concatenate([We1[DN:2 * DN], Wge1[DN:2 * DN], Wn1[DN:2 * DN],
                            Wgn1[DN:2 * DN]], axis=1)
    wstack = jnp.stack([wsrc, wdst])
    w16 = jnp.stack([We1[2 * DN:], Wge1[2 * DN:], We2, Wge2,
                     Wn1[2 * DN:], Wgn1[2 * DN:]])
    b16 = jnp.stack([be1, bge1, be2, bge2, bn1, bgn1])
    w128 = jnp.stack([Wn2, Wgn2])
    b128 = jnp.stack([bn2, bgn2])
    zeros_nd = jnp.zeros((N, DN), f32)

    psrc, pdst = pl.pallas_call(
        _proj_body,
        grid=(N // NB,),
        in_specs=[pl.BlockSpec((NB, DN), lambda i: (i, 0)),
                  pl.BlockSpec((2, DN, 64), lambda i: (0, 0, 0))],
        out_specs=[pl.BlockSpec((NB, 64), lambda i: (i, 0)),
                   pl.BlockSpec((NB, 64), lambda i: (i, 0))],
        out_shape=[jax.ShapeDtypeStruct((N, 64), f32),
                   jax.ShapeDtypeStruct((N, 64), f32)],
    )(node_features, wstack)

    mesh = plsc.VectorSubcoreMesh(core_axis_name="c", subcore_axis_name="s")
    rows_g = ES // 16
    nch_g = rows_g // CG
    rows_s = ES // NW
    nch_s = rows_s // CS
    NT = N // NS

    def make_gather(seg_off):
        @functools.partial(
            pl.kernel,
            out_type=[jax.ShapeDtypeStruct((ES, 64), f32),
                      jax.ShapeDtypeStruct((ES, 64), f32)],
            mesh=mesh,
            scratch_types=[pltpu.VMEM((CG,), jnp.int32),
                           pltpu.VMEM((CG, 64), f32),
                           pltpu.SemaphoreType.DMA],
            compiler_params=pltpu.CompilerParams(use_tc_tiling_on_sc=False),
        )
        def _gather_k(ps_hbm, pd_hbm, src_hbm, dst_hbm, gs_hbm, gd_hbm,
                      idx_v, rows_v, sem):
            cid = lax.axis_index("c")
            sid = lax.axis_index("s")
            wid = sid * NC + cid
            w16_ = wid % 16
            obase = w16_ * rows_g
            ibase = seg_off + obase

            def run(table, idxarr, out):
                def chunk(k, carry):
                    pltpu.sync_copy(idxarr.at[pl.ds(ibase + k * CG, CG)],
                                    idx_v)
                    pltpu.async_copy(table.at[idx_v], rows_v, sem).wait()
                    pltpu.sync_copy(rows_v, out.at[pl.ds(obase + k * CG, CG)])
                    return carry
                lax.fori_loop(0, nch_g, chunk, 0)

            @pl.when(wid < 16)
            def _():
                run(ps_hbm, src_hbm, gs_hbm)

            @pl.when(wid >= 16)
            def _():
                run(pd_hbm, dst_hbm, gd_hbm)

        return _gather_k

    def make_scatter(seg_off):
        @functools.partial(
            pl.kernel,
            out_type=jax.ShapeDtypeStruct((2, N, DN), f32),
            mesh=mesh,
            scratch_types=[pltpu.VMEM((CS,), jnp.int32),
                           pltpu.VMEM((CS, DN), f32),
                           pltpu.VMEM_SHARED((N, DN), f32)],
            compiler_params=pltpu.CompilerParams(use_tc_tiling_on_sc=False),
        )
        def _scatter_k(feat_hbm, src_hbm, z_hbm, out_hbm, idx_v, rows_v, acc):
            cid = lax.axis_index("c")
            sid = lax.axis_index("s")
            wid = sid * NC + cid
            pltpu.sync_copy(z_hbm.at[pl.ds(sid * NT, NT)],
                            acc.at[pl.ds(sid * NT, NT)])
            plsc.subcore_barrier()
            obase = wid * rows_s

            def chunk(k, carry):
                off = obase + k * CS
                pltpu.sync_copy(src_hbm.at[pl.ds(seg_off + off, CS)], idx_v)
                pltpu.sync_copy(feat_hbm.at[pl.ds(off, CS)], rows_v)
                pltpu.sync_copy(rows_v, acc.at[idx_v], add=True)
                return carry
            lax.fori_loop(0, nch_s, chunk, 0)
            plsc.subcore_barrier()
            pltpu.sync_copy(acc.at[pl.ds(sid * NT, NT)],
                            out_hbm.at[cid, pl.ds(sid * NT, NT)])

        return _scatter_k

    full = lambda *s: pl.BlockSpec(s, lambda i: tuple(0 for _ in s))
    eb_per_seg = ES // EB

    ea2_parts = []
    agg_parts = []
    for p in range(SEG):
        off = p * ES
        gs, gd = make_gather(off)(psrc, pdst, src, dst)
        boff = p * eb_per_seg
        ea2_p, feat_p = pl.pallas_call(
            _edge_body,
            grid=(eb_per_seg,),
            in_specs=[pl.BlockSpec((EB, 64), lambda i: (i, 0)),
                      pl.BlockSpec((EB, 64), lambda i: (i, 0)),
                      pl.BlockSpec((EB, DE), lambda i, b=boff: (i + b, 0)),
                      pl.BlockSpec((EB, DEG), lambda i, b=boff: (i + b, 0)),
                      full(6, DE, DE), full(6, DE), full(2, DE, DN),
                      full(2, DN), full(DEG, DE), full(DEG, DN)],
            out_specs=[pl.BlockSpec((EB, DE), lambda i: (i, 0)),
                       pl.BlockSpec((EB, DN), lambda i: (i, 0))],
            out_shape=[jax.ShapeDtypeStruct((ES, DE), f32),
                       jax.ShapeDtypeStruct((ES, DN), f32)],
            compiler_params=pltpu.CompilerParams(
                dimension_semantics=("arbitrary",)),
        )(gs, gd, edge_attr, edge_weights, w16, b16, w128, b128, Wel, Wnl)
        ea2_parts.append(ea2_p)
        agg_parts.append(make_scatter(off)(feat_p, src, zeros_nd))

    ea2 = jnp.concatenate(ea2_parts, axis=0)

    node2 = pl.pallas_call(
        _comb_body,
        grid=(N // NB,),
        in_specs=[pl.BlockSpec((NB, DN), lambda i: (i, 0))] +
                 [pl.BlockSpec((2, NB, DN), lambda i: (0, i, 0))] * SEG,
        out_specs=pl.BlockSpec((NB, DN), lambda i: (i, 0)),
        out_shape=jax.ShapeDtypeStruct((N, DN), f32),
    )(node_features, *agg_parts)

    return (node2, ea2)


# speedup vs baseline: 3.9273x; 1.4543x over previous
import functools

import jax
import jax.numpy as jnp
from jax import lax
from jax.experimental import pallas as pl
from jax.experimental.pallas import tpu as pltpu
from jax.experimental.pallas import tpu_sc as plsc

NC, NS = 2, 16
NW = NC * NS
CG = 1000
CS = 200
NB = 2000
EBF = 800
F = 8

_LOG2E = 1.4426950408889634


def _sig(x):
    return 1.0 / (1.0 + jnp.exp2(x * (-_LOG2E)))


def _silu(x):
    return x * _sig(x)


def _proj_body(nf_ref, w_ref, *out_refs):
    nf = nf_ref[...]
    for r in range(2):
        p = jnp.dot(nf, w_ref[r], preferred_element_type=jnp.float32)
        for h in range(4):
            out_refs[4 * r + h][...] = p[:, 16 * h:16 * (h + 1)]


def _edge_body(gs0, gs1, gs2, gs3, gd0, gd1, gd2, gd3, ea_ref, ew_ref,
               bd16_ref, b16_ref, bdel_ref, w128_ref, b128_ref, wnl_ref,
               ea2_ref, feat_ref):
    def mm(x, w):
        return jnp.dot(x, w, preferred_element_type=jnp.float32)

    ea = ea_ref[...]
    ew = ew_ref[...]
    h1 = _silu(gs0[...] + gd0[...] + mm(ea, bd16_ref[0]) + b16_ref[0])
    g1 = _silu(gs1[...] + gd1[...] + mm(ea, bd16_ref[1]) + b16_ref[1])
    h2 = _silu(mm(h1, bd16_ref[2]) + b16_ref[2])
    g2 = _sig(mm(g1, bd16_ref[3]) + b16_ref[3])
    ea2 = ea + h2 * g2 * mm(ew, bdel_ref[...])
    ea2_ref[...] = ea2
    hn1 = _silu(gs2[...] + gd2[...] + mm(ea2, bd16_ref[4]) + b16_ref[4])
    gn1 = _silu(gs3[...] + gd3[...] + mm(ea2, bd16_ref[5]) + b16_ref[5])
    for k in range(F):
        hk = hn1[:, 16 * k:16 * (k + 1)]
        gk = gn1[:, 16 * k:16 * (k + 1)]
        ek = ew[:, 9 * k:9 * (k + 1)]
        hn2 = _silu(mm(hk, w128_ref[0]) + b128_ref[0])
        gn2 = _sig(mm(gk, w128_ref[1]) + b128_ref[1])
        feat_ref[k] = hn2 * gn2 * mm(ek, wnl_ref[...])


def _comb_body(nf_ref, p_ref, o_ref):
    o_ref[...] = nf_ref[...] + p_ref[0] + p_ref[1]


def kernel(node_features, edge_attr, edge_weights, edge_index, We1, be1, We2,
           be2, Wge1, bge1, Wge2, bge2, Wel, Wn1, bn1, Wn2, bn2, Wgn1, bgn1,
           Wgn2, bgn2, Wnl):
    N, DN = node_features.shape
    E, DE = edge_attr.shape
    DEG = edge_weights.shape[1]
    f32 = jnp.float32
    src = edge_index[0]
    dst = edge_index[1]
    EF = E // F

    wsrc = jnp.concatenate([We1[:DN], Wge1[:DN], Wn1[:DN], Wgn1[:DN]], axis=1)
    wdst = jnp.concatenate([We1[DN:2 * DN], Wge1[DN:2 * DN], Wn1[DN:2 * DN],
                            Wgn1[DN:2 * DN]], axis=1)
    wstack = jnp.stack([wsrc, wdst])
    eye8 = jnp.eye(F, dtype=f32)
    bd16 = jnp.stack([jnp.kron(eye8, W) for W in
                      (We1[2 * DN:], Wge1[2 * DN:], We2, Wge2,
                       Wn1[2 * DN:], Wgn1[2 * DN:])])
    b16 = jnp.stack([jnp.tile(b, F) for b in
                     (be1, bge1, be2, bge2, bn1, bgn1)])
    bdel = jnp.kron(eye8, Wel)
    w128 = jnp.stack([Wn2, Wgn2])
    b128 = jnp.stack([bn2, bgn2])
    zeros_nd = jnp.zeros((N, DN), f32)
    ea_f = edge_attr.reshape(EF, F * DE)
    ew_f = edge_weights.reshape(EF, F * DEG)
    src_perm = src.reshape(EF, F).transpose(1, 0).reshape(E)

    ptabs = pl.pallas_call(
        _proj_body,
        grid=(N // NB,),
        in_specs=[pl.BlockSpec((NB, DN), lambda i: (i, 0)),
                  pl.BlockSpec((2, DN, 64), lambda i: (0, 0, 0))],
        out_specs=[pl.BlockSpec((NB, 16), lambda i: (i, 0))] * 8,
        out_shape=[jax.ShapeDtypeStruct((N, 16), f32)] * 8,
    )(node_features, wstack)

    mesh = plsc.VectorSubcoreMesh(core_axis_name="c", subcore_axis_name="s")
    rows_g = E // 16
    nch_g = rows_g // CG

    @functools.partial(
        pl.kernel,
        out_type=[jax.ShapeDtypeStruct((E, 16), f32)] * 8,
        mesh=mesh,
        scratch_types=[pltpu.VMEM((CG,), jnp.int32)] +
                      [pltpu.VMEM((CG, 16), f32) for _ in range(4)] +
                      [pltpu.SemaphoreType.DMA],
        compiler_params=pltpu.CompilerParams(use_tc_tiling_on_sc=False),
    )
    def _gather_k(t0, t1, t2, t3, t4, t5, t6, t7, src_hbm, dst_hbm,
                  o0, o1, o2, o3, o4, o5, o6, o7,
                  idx_v, r0, r1, r2, r3, sem):
        cid = lax.axis_index("c")
        sid = lax.axis_index("s")
        wid = sid * NC + cid
        base = (wid % 16) * rows_g
        rbufs = (r0, r1, r2, r3)

        def run(tabs, idxarr, outs):
            def chunk(k, carry):
                off = base + k * CG
                pltpu.sync_copy(idxarr.at[pl.ds(off, CG)], idx_v)
                cps = [pltpu.async_copy(tabs[h].at[idx_v], rbufs[h], sem)
                       for h in range(4)]
                for cp in cps:
                    cp.wait()
                for h in range(4):
                    pltpu.sync_copy(rbufs[h], outs[h].at[pl.ds(off, CG)])
                return carry
            lax.fori_loop(0, nch_g, chunk, 0)

        @pl.when(wid < 16)
        def _():
            run((t0, t1, t2, t3), src_hbm, (o0, o1, o2, o3))

        @pl.when(wid >= 16)
        def _():
            run((t4, t5, t6, t7), dst_hbm, (o4, o5, o6, o7))

    g_heads = _gather_k(*ptabs, src, dst)
    gf = [g.reshape(EF, F * 16) for g in g_heads]

    full = lambda *s: pl.BlockSpec(s, lambda i: tuple(0 for _ in s))
    ea2_f, feat8 = pl.pallas_call(
        _edge_body,
        grid=(EF // EBF,),
        in_specs=[pl.BlockSpec((EBF, F * 16), lambda i: (i, 0))] * 8 +
                 [pl.BlockSpec((EBF, F * DE), lambda i: (i, 0)),
                  pl.BlockSpec((EBF, F * DEG), lambda i: (i, 0)),
                  full(6, 128, 128), full(6, 128), full(F * DEG, 128),
                  full(2, DE, DN), full(2, DN), full(DEG, DN)],
        out_specs=[pl.BlockSpec((EBF, F * DE), lambda i: (i, 0)),
                   pl.BlockSpec((F, EBF, DN), lambda i: (0, i, 0))],
        out_shape=[jax.ShapeDtypeStruct((EF, F * DE), f32),
                   jax.ShapeDtypeStruct((F, EF, DN), f32)],
        compiler_params=pltpu.CompilerParams(
            dimension_semantics=("arbitrary",)),
    )(gf[0], gf[1], gf[2], gf[3], gf[4], gf[5], gf[6], gf[7], ea_f, ew_f,
      bd16, b16, bdel, w128, b128, Wnl)

    ea2 = ea2_f.reshape(E, DE)
    feat = feat8.reshape(E, DN)

    rows_s = E // NW
    nch_s = rows_s // CS
    NT = N // NS

    @functools.partial(
        pl.kernel,
        out_type=jax.ShapeDtypeStruct((2, N, DN), f32),
        mesh=mesh,
        scratch_types=[pltpu.VMEM((CS,), jnp.int32),
                       pltpu.VMEM((CS, DN), f32),
                       pltpu.VMEM_SHARED((N, DN), f32)],
        compiler_params=pltpu.CompilerParams(use_tc_tiling_on_sc=False),
    )
    def _scatter_k(feat_hbm, src_hbm, z_hbm, out_hbm, idx_v, rows_v, acc):
        cid = lax.axis_index("c")
        sid = lax.axis_index("s")
        wid = sid * NC + cid
        pltpu.sync_copy(z_hbm.at[pl.ds(sid * NT, NT)],
                        acc.at[pl.ds(sid * NT, NT)])
        plsc.subcore_barrier()
        base = wid * rows_s

        def chunk(k, carry):
            off = base + k * CS
            pltpu.sync_copy(src_hbm.at[pl.ds(off, CS)], idx_v)
            pltpu.sync_copy(feat_hbm.at[pl.ds(off, CS)], rows_v)
            pltpu.sync_copy(rows_v, acc.at[idx_v], add=True)
            return carry
        lax.fori_loop(0, nch_s, chunk, 0)
        plsc.subcore_barrier()
        pltpu.sync_copy(acc.at[pl.ds(sid * NT, NT)],
                        out_hbm.at[cid, pl.ds(sid * NT, NT)])

    parts = _scatter_k(feat, src_perm, zeros_nd)

    node2 = pl.pallas_call(
        _comb_body,
        grid=(N // NB,),
        in_specs=[pl.BlockSpec((NB, DN), lambda i: (i, 0)),
                  pl.BlockSpec((2, NB, DN), lambda i: (0, i, 0))],
        out_specs=pl.BlockSpec((NB, DN), lambda i: (i, 0)),
        out_shape=jax.ShapeDtypeStruct((N, DN), f32),
    )(node_features, parts)

    return (node2, ea2)


# speedup vs baseline: 4.1620x; 1.0598x over previous
import functools

import jax
import jax.numpy as jnp
from jax import lax
from jax.experimental import pallas as pl
from jax.experimental.pallas import tpu as pltpu
from jax.experimental.pallas import tpu_sc as plsc

NC, NS = 2, 16
NW = NC * NS
CG = 1000
CS = 80
NB = 2000
EBF = 800
F = 8

_LOG2E = 1.4426950408889634


def _sig(x):
    return 1.0 / (1.0 + jnp.exp2(x * (-_LOG2E)))


def _silu(x):
    return x * _sig(x)


def _proj_body(nf_ref, w_ref, *out_refs):
    nf = nf_ref[...]
    for r in range(2):
        p = jnp.dot(nf, w_ref[r], preferred_element_type=jnp.float32)
        for h in range(4):
            out_refs[4 * r + h][...] = p[:, 16 * h:16 * (h + 1)]


def _edge_body(gs0, gs1, gs2, gs3, gd0, gd1, gd2, gd3, ea_ref, ew_ref,
               bd16_ref, b16_ref, bdel_ref, w128_ref, b128_ref, wnl_ref,
               ea2_ref, feat_ref):
    def mm(x, w):
        return jnp.dot(x, w, preferred_element_type=jnp.float32)

    ea = ea_ref[...]
    ew = ew_ref[...]
    h1 = _silu(gs0[...] + gd0[...] + mm(ea, bd16_ref[0]) + b16_ref[0])
    g1 = _silu(gs1[...] + gd1[...] + mm(ea, bd16_ref[1]) + b16_ref[1])
    h2 = _silu(mm(h1, bd16_ref[2]) + b16_ref[2])
    g2 = _sig(mm(g1, bd16_ref[3]) + b16_ref[3])
    ea2 = ea + h2 * g2 * mm(ew, bdel_ref[...])
    ea2_ref[...] = ea2
    hn1 = _silu(gs2[...] + gd2[...] + mm(ea2, bd16_ref[4]) + b16_ref[4])
    gn1 = _silu(gs3[...] + gd3[...] + mm(ea2, bd16_ref[5]) + b16_ref[5])
    for k in range(F):
        hk = hn1[:, 16 * k:16 * (k + 1)]
        gk = gn1[:, 16 * k:16 * (k + 1)]
        ek = ew[:, 9 * k:9 * (k + 1)]
        hn2 = _silu(mm(hk, w128_ref[0]) + b128_ref[0])
        gn2 = _sig(mm(gk, w128_ref[1]) + b128_ref[1])
        feat_ref[k] = hn2 * gn2 * mm(ek, wnl_ref[...])


def _comb_body(nf_ref, p_ref, o_ref):
    o_ref[...] = nf_ref[...] + p_ref[0] + p_ref[1]


def kernel(node_features, edge_attr, edge_weights, edge_index, We1, be1, We2,
           be2, Wge1, bge1, Wge2, bge2, Wel, Wn1, bn1, Wn2, bn2, Wgn1, bgn1,
           Wgn2, bgn2, Wnl):
    N, DN = node_features.shape
    E, DE = edge_attr.shape
    DEG = edge_weights.shape[1]
    f32 = jnp.float32
    src = edge_index[0]
    dst = edge_index[1]
    EF = E // F

    wsrc = jnp.concatenate([We1[:DN], Wge1[:DN], Wn1[:DN], Wgn1[:DN]], axis=1)
    wdst = jnp.concatenate([We1[DN:2 * DN], Wge1[DN:2 * DN], Wn1[DN:2 * DN],
                            Wgn1[DN:2 * DN]], axis=1)
    wstack = jnp.stack([wsrc, wdst])
    eye8 = jnp.eye(F, dtype=f32)
    bd16 = jnp.stack([jnp.kron(eye8, W) for W in
                      (We1[2 * DN:], Wge1[2 * DN:], We2, Wge2,
                       Wn1[2 * DN:], Wgn1[2 * DN:])])
    b16 = jnp.stack([jnp.tile(b, F) for b in
                     (be1, bge1, be2, bge2, bn1, bgn1)])
    bdel = jnp.kron(eye8, Wel)
    w128 = jnp.stack([Wn2, Wgn2])
    b128 = jnp.stack([bn2, bgn2])
    zeros_nd = jnp.zeros((N, DN), f32)
    ea_f = edge_attr.reshape(EF, F * DE)
    ew_f = edge_weights.reshape(EF, F * DEG)
    src_perm = src.reshape(EF, F).transpose(1, 0).reshape(E)

    ptabs = pl.pallas_call(
        _proj_body,
        grid=(N // NB,),
        in_specs=[pl.BlockSpec((NB, DN), lambda i: (i, 0)),
                  pl.BlockSpec((2, DN, 64), lambda i: (0, 0, 0))],
        out_specs=[pl.BlockSpec((NB, 16), lambda i: (i, 0))] * 8,
        out_shape=[jax.ShapeDtypeStruct((N, 16), f32)] * 8,
    )(node_features, wstack)

    mesh = plsc.VectorSubcoreMesh(core_axis_name="c", subcore_axis_name="s")
    rows_g = E // 16
    nch_g = rows_g // CG

    @functools.partial(
        pl.kernel,
        out_type=[jax.ShapeDtypeStruct((E, 16), f32)] * 8,
        mesh=mesh,
        scratch_types=[pltpu.VMEM((CG,), jnp.int32)] +
                      [pltpu.VMEM((CG, 16), f32) for _ in range(4)] +
                      [pltpu.SemaphoreType.DMA],
        compiler_params=pltpu.CompilerParams(use_tc_tiling_on_sc=False),
    )
    def _gather_k(t0, t1, t2, t3, t4, t5, t6, t7, src_hbm, dst_hbm,
                  o0, o1, o2, o3, o4, o5, o6, o7,
                  idx_v, r0, r1, r2, r3, sem):
        cid = lax.axis_index("c")
        sid = lax.axis_index("s")
        wid = sid * NC + cid
        base = (wid % 16) * rows_g
        rbufs = (r0, r1, r2, r3)

        def run(tabs, idxarr, outs):
            def chunk(k, carry):
                off = base + k * CG
                pltpu.sync_copy(idxarr.at[pl.ds(off, CG)], idx_v)
                cps = [pltpu.async_copy(tabs[h].at[idx_v], rbufs[h], sem)
                       for h in range(4)]
                for cp in cps:
                    cp.wait()
                for h in range(4):
                    pltpu.sync_copy(rbufs[h], outs[h].at[pl.ds(off, CG)])
                return carry
            lax.fori_loop(0, nch_g, chunk, 0)

        @pl.when(wid < 16)
        def _():
            run((t0, t1, t2, t3), src_hbm, (o0, o1, o2, o3))

        @pl.when(wid >= 16)
        def _():
            run((t4, t5, t6, t7), dst_hbm, (o4, o5, o6, o7))

    g_heads = _gather_k(*ptabs, src, dst)
    gf = [g.reshape(EF, F * 16) for g in g_heads]

    full = lambda *s: pl.BlockSpec(s, lambda i: tuple(0 for _ in s))
    ea2_f, feat8 = pl.pallas_call(
        _edge_body,
        grid=(EF // EBF,),
        in_specs=[pl.BlockSpec((EBF, F * 16), lambda i: (i, 0))] * 8 +
                 [pl.BlockSpec((EBF, F * DE), lambda i: (i, 0)),
                  pl.BlockSpec((EBF, F * DEG), lambda i: (i, 0)),
                  full(6, 128, 128), full(6, 128), full(F * DEG, 128),
                  full(2, DE, DN), full(2, DN), full(DEG, DN)],
        out_specs=[pl.BlockSpec((EBF, F * DE), lambda i: (i, 0)),
                   pl.BlockSpec((F, EBF, DN), lambda i: (0, i, 0))],
        out_shape=[jax.ShapeDtypeStruct((EF, F * DE), f32),
                   jax.ShapeDtypeStruct((F, EF, DN), f32)],
        compiler_params=pltpu.CompilerParams(
            dimension_semantics=("arbitrary",)),
    )(gf[0], gf[1], gf[2], gf[3], gf[4], gf[5], gf[6], gf[7], ea_f, ew_f,
      bd16, b16, bdel, w128, b128, Wnl)

    ea2 = ea2_f.reshape(E, DE)
    feat = feat8.reshape(E, DN)

    rows_s = E // NW
    nch_s = rows_s // CS
    NT = N // NS

    src2 = src_perm.reshape(E // CS, CS)
    rpw = rows_s // CS

    @functools.partial(
        pl.kernel,
        out_type=jax.ShapeDtypeStruct((2, N, DN), f32),
        mesh=mesh,
        scratch_types=[pltpu.VMEM((rows_s // CS, CS), jnp.int32),
                       pltpu.VMEM((CS, DN), f32),
                       pltpu.VMEM((CS, DN), f32),
                       pltpu.SemaphoreType.DMA,
                       pltpu.SemaphoreType.DMA,
                       pltpu.VMEM_SHARED((N, DN), f32)],
        compiler_params=pltpu.CompilerParams(use_tc_tiling_on_sc=False),
    )
    def _scatter_k(feat_hbm, src_hbm, z_hbm, out_hbm, idx_all, rows0,
                   rows1, sem0, sem1, acc):
        cid = lax.axis_index("c")
        sid = lax.axis_index("s")
        wid = sid * NC + cid
        pltpu.sync_copy(src_hbm.at[pl.ds(wid * rpw, rpw)], idx_all)
        pltpu.sync_copy(z_hbm.at[pl.ds(sid * NT, NT)],
                        acc.at[pl.ds(sid * NT, NT)])
        plsc.subcore_barrier()
        base = wid * rows_s
        bufs = ((rows0, sem0), (rows1, sem1))

        def fire(k, b):
            rows_v, sem = bufs[b]
            pltpu.async_copy(feat_hbm.at[pl.ds(base + k * CS, CS)], rows_v,
                             sem)

        def drain_and_add(k, b):
            rows_v, sem = bufs[b]
            pltpu.make_async_copy(feat_hbm.at[pl.ds(base, CS)], rows_v,
                                  sem).wait()
            pltpu.sync_copy(rows_v, acc.at[idx_all.at[k]], add=True)

        fire(0, 0)
        fire(1, 1)

        def pair(t, carry):
            for b in range(2):
                k = 2 * t + b
                drain_and_add(k, b)

                @pl.when(k + 2 < nch_s)
                def _():
                    fire(k + 2, b)
            return carry
        lax.fori_loop(0, nch_s // 2, pair, 0)
        if nch_s % 2:
            drain_and_add(nch_s - 1, 0)
        plsc.subcore_barrier()
        pltpu.sync_copy(acc.at[pl.ds(sid * NT, NT)],
                        out_hbm.at[cid, pl.ds(sid * NT, NT)])

    parts = _scatter_k(feat, src2, zeros_nd)

    node2 = pl.pallas_call(
        _comb_body,
        grid=(N // NB,),
        in_specs=[pl.BlockSpec((NB, DN), lambda i: (i, 0)),
                  pl.BlockSpec((2, NB, DN), lambda i: (0, i, 0))],
        out_specs=pl.BlockSpec((NB, DN), lambda i: (i, 0)),
        out_shape=jax.ShapeDtypeStruct((N, DN), f32),
    )(node_features, parts)

    return (node2, ea2)
